# Initial kernel scaffold; baseline (speedup 1.0000x reference)
import jax
import jax.numpy as jnp
from jax.experimental import pallas as pl


def kernel(words, table):
    raise NotImplementedError("write your pallas kernel here")


# speedup vs baseline: 1.9309x; 1.9309x over previous
import functools

import jax
import jax.numpy as jnp
import numpy as np
from jax import lax
from jax.experimental import pallas as pl
from jax.experimental.pallas import tpu as pltpu
from jax.experimental.pallas import tpu_sc as plsc

VOCAB_DIM = 128
SPAN = 10
LANES = 16
NUM_WORKERS = 32
CHUNK_B = 8
CHUNK_IDX = CHUNK_B * SPAN


def _cbow_body(words_hbm, table_hbm, out_hbm, idx_v, rows_v, out_v, sem):
    b_per_w = out_hbm.shape[0] // NUM_WORKERS
    n_steps = b_per_w // CHUNK_B
    wid = lax.axis_index("s") * 2 + lax.axis_index("c")
    pltpu.sync_copy(words_hbm.at[pl.ds(wid * n_steps, n_steps)], idx_v)

    def step(s, _):
        pltpu.async_copy(table_hbm.at[idx_v.at[s]], rows_v, sem).wait()
        inv_span = jnp.float32(1.0 / SPAN)
        for b in range(CHUNK_B):
            for g in range(VOCAB_DIM // LANES):
                sl = pl.ds(g * LANES, LANES)
                acc = rows_v[b * SPAN, sl]
                for j in range(1, SPAN):
                    acc = acc + rows_v[b * SPAN + j, sl]
                out_v[b, sl] = acc * inv_span
        pltpu.sync_copy(
            out_v, out_hbm.at[pl.ds(wid * b_per_w + s * CHUNK_B, CHUNK_B)]
        )
        return 0

    lax.fori_loop(0, n_steps, step, 0)


def kernel(words, table):
    batch, span = words.shape
    assert span == SPAN and table.shape[1] == VOCAB_DIM
    words2d = words.reshape(batch * span // CHUNK_IDX, CHUNK_IDX)

    mesh = plsc.VectorSubcoreMesh(core_axis_name="c", subcore_axis_name="s")
    f = pl.kernel(
        _cbow_body,
        out_type=jax.ShapeDtypeStruct((batch, VOCAB_DIM), jnp.float32),
        mesh=mesh,
        scratch_types=[
            pltpu.VMEM((batch * span // CHUNK_IDX // NUM_WORKERS, CHUNK_IDX), jnp.int32),
            pltpu.VMEM((CHUNK_IDX, VOCAB_DIM), jnp.float32),
            pltpu.VMEM((CHUNK_B, VOCAB_DIM), jnp.float32),
            pltpu.SemaphoreType.DMA,
        ],
    )
    return f(words2d, table)


# speedup vs baseline: 2.9889x; 1.5479x over previous
import jax
import jax.numpy as jnp
from jax import lax
from jax.experimental import pallas as pl
from jax.experimental.pallas import tpu as pltpu
from jax.experimental.pallas import tpu_sc as plsc

DIM = 128
SPAN = 10
LANES = 16
NUM_WORKERS = 32
CHUNK_B = 8
K_FIRE = 4
MACRO_B = CHUNK_B * K_FIRE


def _cbow_body(words_hbm, table_hbm, out_hbm, idx_v, rows0, rows1, out_v,
               sem0, sem1):
    b_per_w = out_hbm.shape[0] // NUM_WORKERS
    n_macro = b_per_w // MACRO_B
    n_chunks = b_per_w // CHUNK_B
    wid = lax.axis_index("s") * 2 + lax.axis_index("c")
    pltpu.sync_copy(words_hbm.at[pl.ds(wid * n_chunks, n_chunks)], idx_v)

    def fire(s, buf, sem):
        for k in range(K_FIRE):
            pltpu.async_copy(
                table_hbm.at[idx_v.at[s * K_FIRE + k]],
                buf.at[pl.ds(k * CHUNK_B * SPAN, CHUNK_B * SPAN)],
                sem,
            )

    def drain(buf, sem):
        for k in range(K_FIRE):
            pltpu.make_async_copy(
                table_hbm.at[idx_v.at[0]],
                buf.at[pl.ds(k * CHUNK_B * SPAN, CHUNK_B * SPAN)],
                sem,
            ).wait()

    def compute(s, buf):
        inv_span = jnp.float32(1.0 / SPAN)

        def one_row(b, _):
            base = b * SPAN
            for g in range(DIM // LANES):
                sl = pl.ds(g * LANES, LANES)
                acc = buf[base, sl]
                for j in range(1, SPAN):
                    acc = acc + buf[base + j, sl]
                out_v[b, sl] = acc * inv_span
            return 0

        lax.fori_loop(0, MACRO_B, one_row, 0)
        pltpu.sync_copy(
            out_v, out_hbm.at[pl.ds(wid * b_per_w + s * MACRO_B, MACRO_B)]
        )

    fire(0, rows0, sem0)

    def pair(i, _):
        s0 = 2 * i
        fire(s0 + 1, rows1, sem1)
        drain(rows0, sem0)
        compute(s0, rows0)

        s1 = 2 * i + 1

        @pl.when(i < n_macro // 2 - 1)
        def _():
            fire(s1 + 1, rows0, sem0)

        drain(rows1, sem1)
        compute(s1, rows1)
        return 0

    lax.fori_loop(0, n_macro // 2, pair, 0)


def kernel(words, table):
    batch, span = words.shape
    assert span == SPAN and table.shape[1] == DIM
    n_chunks_total = batch * span // (CHUNK_B * SPAN)
    words2d = words.reshape(n_chunks_total, CHUNK_B * SPAN)

    mesh = plsc.VectorSubcoreMesh(core_axis_name="c", subcore_axis_name="s")
    f = pl.kernel(
        _cbow_body,
        out_type=jax.ShapeDtypeStruct((batch, DIM), jnp.float32),
        mesh=mesh,
        scratch_types=[
            pltpu.VMEM((n_chunks_total // NUM_WORKERS, CHUNK_B * SPAN), jnp.int32),
            pltpu.VMEM((MACRO_B * SPAN, DIM), jnp.float32),
            pltpu.VMEM((MACRO_B * SPAN, DIM), jnp.float32),
            pltpu.VMEM((MACRO_B, DIM), jnp.float32),
            pltpu.SemaphoreType.DMA,
            pltpu.SemaphoreType.DMA,
        ],
    )
    return f(words2d, table)


# speedup vs baseline: 4.2353x; 1.4170x over previous
import jax
import jax.numpy as jnp
from jax import lax
from jax.experimental import pallas as pl
from jax.experimental.pallas import tpu as pltpu
from jax.experimental.pallas import tpu_sc as plsc

DIM = 128
SPAN = 10
LANES = 16
NUM_WORKERS = 32
CHUNK_B = 8
K_FIRE = 4
MACRO_B = CHUNK_B * K_FIRE


def _cbow_body(words_hbm, table_hbm, out_hbm, idx_v, rows0, rows1, out0,
               out1, sem0, sem1, sem_o0, sem_o1):
    b_per_w = out_hbm.shape[0] // NUM_WORKERS
    n_macro = b_per_w // MACRO_B
    n_chunks = b_per_w // CHUNK_B
    wid = lax.axis_index("s") * 2 + lax.axis_index("c")
    pltpu.sync_copy(words_hbm.at[pl.ds(wid * n_chunks, n_chunks)], idx_v)

    def fire(s, buf, sem):
        for k in range(K_FIRE):
            pltpu.async_copy(
                table_hbm.at[idx_v.at[s * K_FIRE + k]],
                buf.at[pl.ds(k * CHUNK_B * SPAN, CHUNK_B * SPAN)],
                sem,
            )

    def drain(buf, sem):
        for k in range(K_FIRE):
            pltpu.make_async_copy(
                table_hbm.at[idx_v.at[0]],
                buf.at[pl.ds(k * CHUNK_B * SPAN, CHUNK_B * SPAN)],
                sem,
            ).wait()

    def out_slice(s):
        return out_hbm.at[pl.ds(wid * b_per_w + s * MACRO_B, MACRO_B)]

    def compute(s, buf, out_v, sem_o):
        inv_span = jnp.float32(1.0 / SPAN)

        @plsc.parallel_loop(0, MACRO_B, unroll=2)
        def _(b):
            base = b * SPAN
            for g in range(DIM // LANES):
                sl = pl.ds(g * LANES, LANES)
                acc = buf[base, sl]
                for j in range(1, SPAN):
                    acc = acc + buf[base + j, sl]
                out_v[b, sl] = acc * inv_span

        pltpu.async_copy(out_v, out_slice(s), sem_o)

    fire(0, rows0, sem0)

    def pair(i, _):
        s0 = 2 * i
        fire(s0 + 1, rows1, sem1)
        drain(rows0, sem0)

        @pl.when(i > 0)
        def _():
            pltpu.make_async_copy(out0, out_slice(0), sem_o0).wait()

        compute(s0, rows0, out0, sem_o0)

        s1 = 2 * i + 1

        @pl.when(i < n_macro // 2 - 1)
        def _():
            fire(s1 + 1, rows0, sem0)

        drain(rows1, sem1)

        @pl.when(i > 0)
        def _():
            pltpu.make_async_copy(out1, out_slice(0), sem_o1).wait()

        compute(s1, rows1, out1, sem_o1)
        return 0

    lax.fori_loop(0, n_macro // 2, pair, 0)
    pltpu.make_async_copy(out0, out_slice(0), sem_o0).wait()
    pltpu.make_async_copy(out1, out_slice(0), sem_o1).wait()


def kernel(words, table):
    batch, span = words.shape
    assert span == SPAN and table.shape[1] == DIM
    n_chunks_total = batch * span // (CHUNK_B * SPAN)
    words2d = words.reshape(n_chunks_total, CHUNK_B * SPAN)

    mesh = plsc.VectorSubcoreMesh(core_axis_name="c", subcore_axis_name="s")
    f = pl.kernel(
        _cbow_body,
        out_type=jax.ShapeDtypeStruct((batch, DIM), jnp.float32),
        mesh=mesh,
        scratch_types=[
            pltpu.VMEM((n_chunks_total // NUM_WORKERS, CHUNK_B * SPAN), jnp.int32),
            pltpu.VMEM((MACRO_B * SPAN, DIM), jnp.float32),
            pltpu.VMEM((MACRO_B * SPAN, DIM), jnp.float32),
            pltpu.VMEM((MACRO_B, DIM), jnp.float32),
            pltpu.VMEM((MACRO_B, DIM), jnp.float32),
            pltpu.SemaphoreType.DMA,
            pltpu.SemaphoreType.DMA,
            pltpu.SemaphoreType.DMA,
            pltpu.SemaphoreType.DMA,
        ],
    )
    return f(words2d, table)


# speedup vs baseline: 4.3704x; 1.0319x over previous
import jax
import jax.numpy as jnp
from jax import lax
from jax.experimental import pallas as pl
from jax.experimental.pallas import tpu as pltpu
from jax.experimental.pallas import tpu_sc as plsc

DIM = 128
SPAN = 10
LANES = 16
NUM_WORKERS = 32
CHUNK_B = 8
K_FIRE = 4
MACRO_B = CHUNK_B * K_FIRE


def _cbow_body(words_hbm, table_hbm, out_hbm, idx_v, rows0, rows1, out0,
               out1, sem0, sem1, sem_o0, sem_o1):
    b_per_w = out_hbm.shape[0] // NUM_WORKERS
    n_macro = b_per_w // MACRO_B
    n_chunks = b_per_w // CHUNK_B
    wid = lax.axis_index("s") * 2 + lax.axis_index("c")
    pltpu.sync_copy(words_hbm.at[pl.ds(wid * n_chunks, n_chunks)], idx_v)

    def fire(s, buf, sem):
        for k in range(K_FIRE):
            pltpu.async_copy(
                table_hbm.at[idx_v.at[s * K_FIRE + k]],
                buf.at[pl.ds(k * CHUNK_B * SPAN, CHUNK_B * SPAN)],
                sem,
            )

    def drain(buf, sem):
        for k in range(K_FIRE):
            pltpu.make_async_copy(
                table_hbm.at[idx_v.at[0]],
                buf.at[pl.ds(k * CHUNK_B * SPAN, CHUNK_B * SPAN)],
                sem,
            ).wait()

    def out_slice(s):
        return out_hbm.at[pl.ds(wid * b_per_w + s * MACRO_B, MACRO_B)]

    def compute(s, buf, out_v, sem_o):
        inv_span = jnp.float32(1.0 / SPAN)

        @plsc.parallel_loop(0, MACRO_B, unroll=4)
        def _(b):
            base = b * SPAN
            for g in range(DIM // LANES):
                sl = pl.ds(g * LANES, LANES)
                acc = buf[base, sl]
                for j in range(1, SPAN):
                    acc = acc + buf[base + j, sl]
                out_v[b, sl] = acc * inv_span

        pltpu.async_copy(out_v, out_slice(s), sem_o)

    fire(0, rows0, sem0)

    def pair(i, _):
        s0 = 2 * i
        fire(s0 + 1, rows1, sem1)
        drain(rows0, sem0)

        @pl.when(i > 0)
        def _():
            pltpu.make_async_copy(out0, out_slice(0), sem_o0).wait()

        compute(s0, rows0, out0, sem_o0)

        s1 = 2 * i + 1

        @pl.when(i < n_macro // 2 - 1)
        def _():
            fire(s1 + 1, rows0, sem0)

        drain(rows1, sem1)

        @pl.when(i > 0)
        def _():
            pltpu.make_async_copy(out1, out_slice(0), sem_o1).wait()

        compute(s1, rows1, out1, sem_o1)
        return 0

    lax.fori_loop(0, n_macro // 2, pair, 0)
    pltpu.make_async_copy(out0, out_slice(0), sem_o0).wait()
    pltpu.make_async_copy(out1, out_slice(0), sem_o1).wait()


def kernel(words, table):
    batch, span = words.shape
    assert span == SPAN and table.shape[1] == DIM
    n_chunks_total = batch * span // (CHUNK_B * SPAN)
    words2d = words.reshape(n_chunks_total, CHUNK_B * SPAN)

    mesh = plsc.VectorSubcoreMesh(core_axis_name="c", subcore_axis_name="s")
    f = pl.kernel(
        _cbow_body,
        out_type=jax.ShapeDtypeStruct((batch, DIM), jnp.float32),
        mesh=mesh,
        scratch_types=[
            pltpu.VMEM((n_chunks_total // NUM_WORKERS, CHUNK_B * SPAN), jnp.int32),
            pltpu.VMEM((MACRO_B * SPAN, DIM), jnp.float32),
            pltpu.VMEM((MACRO_B * SPAN, DIM), jnp.float32),
            pltpu.VMEM((MACRO_B, DIM), jnp.float32),
            pltpu.VMEM((MACRO_B, DIM), jnp.float32),
            pltpu.SemaphoreType.DMA,
            pltpu.SemaphoreType.DMA,
            pltpu.SemaphoreType.DMA,
            pltpu.SemaphoreType.DMA,
        ],
    )
    return f(words2d, table)
